# Initial kernel scaffold; baseline (speedup 1.0000x reference)
import jax
import jax.numpy as jnp
from jax.experimental import pallas as pl


def kernel(indices, w_1st, w_2nd):
    raise NotImplementedError("write your pallas kernel here")


# speedup vs baseline: 1.2258x; 1.2258x over previous
import functools

import jax
import jax.numpy as jnp
from jax import lax
from jax.experimental import pallas as pl
from jax.experimental.pallas import tpu as pltpu
from jax.experimental.pallas import tpu_sc as plsc

B = 16384
F = 26
M = 16

_NC = 2
_NS = 16
_NW = _NC * _NS
_SPW = B // _NW
_C = 128
_NCHUNK = _SPW // _C
_IPC = _C * F
_GL = 128
_G = _IPC // _GL


def _fm_body(idx_hbm, w1_hbm, w2_hbm, out_hbm,
             idx_v, w1_v, rows_v, out_v, tbuf, sem1, sem2):
    cid = lax.axis_index("c")
    sid = lax.axis_index("s")
    wid = sid * _NC + cid
    base = wid * _SPW

    def chunk_body(c, carry):
        off = (base + c * _C) * F
        pltpu.sync_copy(idx_hbm.at[pl.ds(off, _IPC)], idx_v)
        cps = []
        for j in range(_G):
            sl = pl.ds(j * _GL, _GL)
            cps.append(pltpu.async_copy(w2_hbm.at[idx_v.at[sl]],
                                        rows_v.at[sl], sem2))
            cps.append(pltpu.async_copy(w1_hbm.at[idx_v.at[sl]],
                                        w1_v.at[sl], sem1))
        for cp in cps:
            cp.wait()

        lanes = lax.iota(jnp.int32, 16)

        def group_body(g, carry2):
            def sample_body(s, carry3):
                rbase = (g * 16 + s) * F
                r0 = rows_v[rbase]
                acc_s = r0
                acc_q = r0 * r0
                for f in range(1, F):
                    r = rows_v[rbase + f]
                    acc_s = acc_s + r
                    acc_q = acc_q + r * r
                tbuf[pl.ds(s * 16, 16)] = acc_s * acc_s - acc_q
                return carry3

            lax.fori_loop(0, 16, sample_body, 0)

            lbase = g * (16 * F) + lanes * F
            lin = plsc.load_gather(w1_v, [lbase])
            for f in range(1, F):
                lin = lin + plsc.load_gather(w1_v, [lbase + f])

            pbase = lanes * 16
            pair = plsc.load_gather(tbuf, [pbase])
            for m in range(1, 16):
                pair = pair + plsc.load_gather(tbuf, [pbase + m])

            out_v[pl.ds(g * 16, 16)] = lin + 0.5 * pair
            return carry2

        lax.fori_loop(0, _C // 16, group_body, 0)
        pltpu.sync_copy(out_v, out_hbm.at[pl.ds(base + c * _C, _C)])
        return carry

    lax.fori_loop(0, _NCHUNK, chunk_body, 0)


@jax.jit
def kernel(indices, w_1st, w_2nd):
    idx_flat = indices.reshape(-1).astype(jnp.int32)
    mesh = plsc.VectorSubcoreMesh(core_axis_name="c", subcore_axis_name="s")
    fm = pl.kernel(
        _fm_body,
        out_type=jax.ShapeDtypeStruct((B,), jnp.float32),
        mesh=mesh,
        compiler_params=pltpu.CompilerParams(
            needs_layout_passes=False, use_tc_tiling_on_sc=False),
        scratch_types=[
            pltpu.VMEM((_IPC,), jnp.int32),
            pltpu.VMEM((_IPC,), jnp.float32),
            pltpu.VMEM((_IPC, M), jnp.float32),
            pltpu.VMEM((_C,), jnp.float32),
            pltpu.VMEM((256,), jnp.float32),
            pltpu.SemaphoreType.DMA,
            pltpu.SemaphoreType.DMA,
        ],
    )
    return fm(idx_flat, w_1st, w_2nd)
